# Initial kernel scaffold; baseline (speedup 1.0000x reference)
import functools

import jax
import jax.numpy as jnp
from jax import lax
from jax.experimental import pallas as pl
from jax.experimental.pallas import tpu as pltpu
from jax.experimental.pallas import tpu_sc as plsc

NU, N1, N2, DIN, H, E = 4096, 8192, 8192, 128, 64, 131072
S1, S2 = 8, 4
TAU, LAM = 0.8, 0.5

NC, NS = 2, 16
NW = NC * NS
EPT = E // NW
HP = 80
UFW = 132


def _elu(x):
    return jnp.where(x > 0, x, jnp.expm1(x))


def _proj_body(x_ref, wm_ref, bm_ref, wr_ref, o_ref):
    h = _elu(jnp.dot(x_ref[...], wm_ref[...], preferred_element_type=jnp.float32)
             + bm_ref[...])
    o_ref[...] = jnp.dot(h, wr_ref[...], preferred_element_type=jnp.float32)


def _proj(x, Wmap, bmap, Wright, br=512):
    n = x.shape[0]
    co = Wright.shape[1]
    return pl.pallas_call(
        _proj_body,
        grid=(n // br,),
        in_specs=[
            pl.BlockSpec((br, DIN), lambda i: (i, 0)),
            pl.BlockSpec((DIN, H), lambda i: (0, 0)),
            pl.BlockSpec((1, H), lambda i: (0, 0)),
            pl.BlockSpec((H, co), lambda i: (0, 0)),
        ],
        out_specs=pl.BlockSpec((br, co), lambda i: (i, 0)),
        out_shape=jax.ShapeDtypeStruct((n, co), jnp.float32),
    )(x, Wmap, bmap.reshape(1, H), Wright)


def _sc_deg(dst0, dst1):
    mesh = plsc.VectorSubcoreMesh(core_axis_name="c", subcore_axis_name="s")

    @functools.partial(
        pl.kernel, mesh=mesh,
        out_type=jax.ShapeDtypeStruct((NC, 2, 256, 16), jnp.float32),
        scratch_types=[
            pltpu.VMEM((EPT,), jnp.int32),
            pltpu.VMEM((256, 16), jnp.float32),
            pltpu.VMEM((256, 16), jnp.float32),
            pltpu.VMEM((256,), jnp.int32),
            pltpu.VMEM((16, 16), jnp.float32),
            pltpu.VMEM_SHARED((256, 16), jnp.float32),
            pltpu.VMEM_SHARED((256, 16), jnp.float32),
        ],
    )
    def k(d0_hbm, d1_hbm, degp, dstv, h0, h1, rowidx, zb, sh0, sh1):
        cid = lax.axis_index("c")
        sid = lax.axis_index("s")
        wid = cid * NS + sid
        z16 = jnp.zeros((16,), jnp.float32)
        for r in range(16):
            zb[r, :] = z16

        def fz(i, _):
            h0[i, :] = z16
            h1[i, :] = z16
            return 0
        lax.fori_loop(0, 256, fz, 0)

        def fr(i, _):
            rowidx[pl.ds(i * 16, 16)] = lax.iota(jnp.int32, 16) + i * 16
            return 0
        lax.fori_loop(0, 16, fr, 0)

        pltpu.sync_copy(zb, sh0.at[pl.ds(sid * 16, 16), :])
        pltpu.sync_copy(zb, sh1.at[pl.ds(sid * 16, 16), :])
        plsc.subcore_barrier()

        ones = jnp.full((16,), 1.0, jnp.float32)
        for mp in range(2):
            d_hbm, hist = ((d0_hbm, h0), (d1_hbm, h1))[mp]
            pltpu.sync_copy(d_hbm.at[pl.ds(wid * EPT, EPT)], dstv)

            def fh(i, _):
                idx = dstv[pl.ds(i * 16, 16)]
                plsc.addupdate_scatter(
                    hist,
                    [lax.shift_right_logical(idx, 4), lax.bitwise_and(idx, 15)],
                    ones)
                return 0
            lax.fori_loop(0, 256, fh, 0)

        pltpu.sync_copy(h0, sh0.at[rowidx], add=True)
        pltpu.sync_copy(h1, sh1.at[rowidx], add=True)
        plsc.subcore_barrier()

        @pl.when(sid == 0)
        def _():
            pltpu.sync_copy(sh0, degp.at[cid, 0])
            pltpu.sync_copy(sh1, degp.at[cid, 1])

    return k(dst0, dst1)


def _sc_segsum(y0, y1, src0, dst0, src1, dst1):
    mesh = plsc.VectorSubcoreMesh(core_axis_name="c", subcore_axis_name="s")
    cpt = EPT // 128

    @functools.partial(
        pl.kernel, mesh=mesh,
        out_type=jax.ShapeDtypeStruct((NC, 2, NU, H), jnp.float32),
        scratch_types=[
            pltpu.VMEM((cpt, 128), jnp.int32),
            pltpu.VMEM((cpt, 128), jnp.int32),
            pltpu.VMEM((128, H), jnp.float32),
            pltpu.VMEM((256, H), jnp.float32),
            pltpu.VMEM_SHARED((NU, H), jnp.float32),
            pltpu.SemaphoreType.DMA,
        ],
    )
    def k(y0_hbm, y1_hbm, s0_hbm, d0_hbm, s1_hbm, d1_hbm, gout,
          srcv, dstv, rows, zbuf, acc, sem):
        cid = lax.axis_index("c")
        sid = lax.axis_index("s")
        wid = cid * NS + sid
        z16 = jnp.zeros((16,), jnp.float32)

        def fz(i, _):
            for c in range(H // 16):
                zbuf[i, pl.ds(c * 16, 16)] = z16
            return 0
        lax.fori_loop(0, 256, fz, 0)

        for mp in range(2):
            y_hbm, s_hbm, d_hbm = ((y0_hbm, s0_hbm, d0_hbm),
                                   (y1_hbm, s1_hbm, d1_hbm))[mp]
            pltpu.sync_copy(zbuf, acc.at[pl.ds(sid * 256, 256), :])
            plsc.subcore_barrier()
            pltpu.sync_copy(s_hbm.at[pl.ds(wid * cpt, cpt), :], srcv)
            pltpu.sync_copy(d_hbm.at[pl.ds(wid * cpt, cpt), :], dstv)

            def chunk(j, _):
                pltpu.async_copy(y_hbm.at[srcv.at[j]], rows, sem).wait()
                pltpu.sync_copy(rows, acc.at[dstv.at[j]], add=True)
                return 0
            lax.fori_loop(0, cpt, chunk, 0)
            plsc.subcore_barrier()
            pltpu.sync_copy(acc.at[pl.ds(sid * 256, 256), :],
                            gout.at[cid, mp, pl.ds(sid * 256, 256), :])
            plsc.subcore_barrier()

    return k(y0, y1, src0, dst0, src1, dst1)


def _sc_intra(hp, neif, uf, S, tcol):
    mesh = plsc.VectorSubcoreMesh(core_axis_name="c", subcore_axis_name="s")
    P = NU // NW

    @functools.partial(
        pl.kernel, mesh=mesh,
        out_type=jax.ShapeDtypeStruct((NU, H), jnp.float32),
        scratch_types=[
            pltpu.VMEM((S, 128), jnp.int32),
            pltpu.VMEM((P * S, HP), jnp.float32),
            pltpu.VMEM((P, UFW), jnp.float32),
            pltpu.VMEM((P, H), jnp.float32),
            pltpu.SemaphoreType.DMA,
        ],
    )
    def k(hp_hbm, nei_hbm, uf_hbm, sout, neiv, rows, ccv, outv, sem):
        cid = lax.axis_index("c")
        sid = lax.axis_index("s")
        wid = cid * NS + sid
        pltpu.sync_copy(nei_hbm.at[pl.ds(wid * S, S), :], neiv)
        pltpu.sync_copy(uf_hbm.at[pl.ds(wid * P, P), :], ccv)
        cps = [pltpu.async_copy(hp_hbm.at[neiv.at[j]],
                                rows.at[pl.ds(j * 128, 128), :], sem)
               for j in range(S)]
        for cp in cps:
            cp.wait()

        lane = lax.iota(jnp.int32, 16)
        kk = jnp.minimum(lane, S - 1)
        col_d = jnp.full((16,), H, jnp.int32)
        col_c = jnp.full((16,), 2 * H + tcol, jnp.int32)

        def ub(u, _):
            base = u * S
            dv = plsc.load_gather(rows, [base + kk, col_d])
            cu = plsc.load_gather(ccv, [lane * 0 + u, col_c])
            l = cu + dv
            l = jnp.where(l > 0, l, 0.01 * l)
            l = jnp.where(lane < S, l, -1e30)
            p = jnp.exp(l - jnp.max(l))
            den = jnp.sum(p)
            ws = [p[kq] for kq in range(S)]
            for f in range(H // 16):
                acc = rows[base, pl.ds(f * 16, 16)] * ws[0]
                for kq in range(1, S):
                    acc = acc + rows[base + kq, pl.ds(f * 16, 16)] * ws[kq]
                outv[u, pl.ds(f * 16, 16)] = acc / den
            return 0
        lax.fori_loop(0, P, ub, 0)
        pltpu.sync_copy(outv, sout.at[pl.ds(wid * P, P), :])

    return k(hp, neif, uf)


def _y_body(deg_ref, uf_ref, y0_ref, y1_ref, di_ref):
    dinv = lax.rsqrt(deg_ref[...] + 1.0)
    di_ref[...] = dinv
    y0_ref[...] = dinv[:, 0:1] * uf_ref[:, 0:H]
    y1_ref[...] = dinv[:, 1:2] * uf_ref[:, H:2 * H]


def _yk(deg2, uf):
    br = 512
    return pl.pallas_call(
        _y_body,
        grid=(NU // br,),
        in_specs=[
            pl.BlockSpec((br, 2), lambda i: (i, 0)),
            pl.BlockSpec((br, UFW), lambda i: (i, 0)),
        ],
        out_specs=[
            pl.BlockSpec((br, H), lambda i: (i, 0)),
            pl.BlockSpec((br, H), lambda i: (i, 0)),
            pl.BlockSpec((br, 2), lambda i: (i, 0)),
        ],
        out_shape=[
            jax.ShapeDtypeStruct((NU, H), jnp.float32),
            jax.ShapeDtypeStruct((NU, H), jnp.float32),
            jax.ShapeDtypeStruct((NU, 2), jnp.float32),
        ],
    )(deg2, uf)


def _e_body(g_ref, y0_ref, y1_ref, di_ref, bg_ref, al_ref, e0_ref, e1_ref):
    g = g_ref[...]
    di = di_ref[...]
    al = al_ref[...]
    bg = bg_ref[...]
    t0 = di[:, 0:1] * (g[0, 0] + g[1, 0] + y0_ref[...]) + bg[0:1, :]
    e0_ref[...] = jnp.where(t0 > 0, t0, al[0:1, 0:1] * t0)
    t1 = di[:, 1:2] * (g[0, 1] + g[1, 1] + y1_ref[...]) + bg[1:2, :]
    e1_ref[...] = jnp.where(t1 > 0, t1, al[0:1, 1:2] * t1)


def _ek(gout, y0, y1, di, bg, al):
    br = 512
    return pl.pallas_call(
        _e_body,
        grid=(NU // br,),
        in_specs=[
            pl.BlockSpec((NC, 2, br, H), lambda i: (0, 0, i, 0)),
            pl.BlockSpec((br, H), lambda i: (i, 0)),
            pl.BlockSpec((br, H), lambda i: (i, 0)),
            pl.BlockSpec((br, 2), lambda i: (i, 0)),
            pl.BlockSpec((2, H), lambda i: (0, 0)),
            pl.BlockSpec((1, 2), lambda i: (0, 0)),
        ],
        out_specs=[
            pl.BlockSpec((br, H), lambda i: (i, 0)),
            pl.BlockSpec((br, H), lambda i: (i, 0)),
        ],
        out_shape=[
            jax.ShapeDtypeStruct((NU, H), jnp.float32),
            jax.ShapeDtypeStruct((NU, H), jnp.float32),
        ],
    )(gout, y0, y1, di, bg, al)


def _sem_body(e0_ref, e1_ref, s0r_ref, s1r_ref, wmp_ref, bmp_ref,
              wsc_ref, bsc_ref, s0_ref, s1_ref, ts_ref):
    i = pl.program_id(0)
    s0 = _elu(s0r_ref[...])
    s1 = _elu(s1r_ref[...])
    s0_ref[...] = s0
    s1_ref[...] = s1
    wmp = wmp_ref[...]
    bmp = bmp_ref[...]
    wsc = wsc_ref[...]
    bsc = bsc_ref[...]
    r0 = jnp.sum(jnp.tanh(jnp.dot(e0_ref[...], wmp,
                                  preferred_element_type=jnp.float32) + bmp),
                 axis=0, keepdims=True)
    r1 = jnp.sum(jnp.tanh(jnp.dot(e1_ref[...], wmp,
                                  preferred_element_type=jnp.float32) + bmp),
                 axis=0, keepdims=True)
    r2 = jnp.sum(jnp.tanh(jnp.dot(s0, wsc,
                                  preferred_element_type=jnp.float32) + bsc),
                 axis=0, keepdims=True)
    r3 = jnp.sum(jnp.tanh(jnp.dot(s1, wsc,
                                  preferred_element_type=jnp.float32) + bsc),
                 axis=0, keepdims=True)
    blk = jnp.concatenate([r0, r1, r2, r3, jnp.zeros((4, H), jnp.float32)], 0)

    @pl.when(i == 0)
    def _():
        ts_ref[...] = jnp.zeros((8, H), jnp.float32)

    ts_ref[...] += blk


def _semk(e0, e1, s0r, s1r, wmp, bmp, wsc, bsc):
    br = 512
    return pl.pallas_call(
        _sem_body,
        grid=(NU // br,),
        in_specs=[
            pl.BlockSpec((br, H), lambda i: (i, 0)),
            pl.BlockSpec((br, H), lambda i: (i, 0)),
            pl.BlockSpec((br, H), lambda i: (i, 0)),
            pl.BlockSpec((br, H), lambda i: (i, 0)),
            pl.BlockSpec((H, H), lambda i: (0, 0)),
            pl.BlockSpec((1, H), lambda i: (0, 0)),
            pl.BlockSpec((H, H), lambda i: (0, 0)),
            pl.BlockSpec((1, H), lambda i: (0, 0)),
        ],
        out_specs=[
            pl.BlockSpec((br, H), lambda i: (i, 0)),
            pl.BlockSpec((br, H), lambda i: (i, 0)),
            pl.BlockSpec((8, H), lambda i: (0, 0)),
        ],
        out_shape=[
            jax.ShapeDtypeStruct((NU, H), jnp.float32),
            jax.ShapeDtypeStruct((NU, H), jnp.float32),
            jax.ShapeDtypeStruct((8, H), jnp.float32),
        ],
    )(e0, e1, s0r, s1r, wmp, bmp.reshape(1, H), wsc, bsc.reshape(1, H))


def _zp_body(e0_ref, e1_ref, s0_ref, s1_ref, bc_ref, w1_ref, b1_ref,
             w2_ref, b2_ref, znm_ref, zns_ref):
    bc = bc_ref[...]
    w1 = w1_ref[...]
    b1 = b1_ref[...]
    w2 = w2_ref[...]
    b2 = b2_ref[...]

    def proj(z):
        zp = jnp.dot(_elu(jnp.dot(z, w1, preferred_element_type=jnp.float32)
                          + b1), w2, preferred_element_type=jnp.float32) + b2
        return zp * lax.rsqrt(jnp.sum(zp * zp, axis=1, keepdims=True))

    znm_ref[...] = proj(bc[0:1, 0:1] * e0_ref[...] + bc[0:1, 1:2] * e1_ref[...])
    zns_ref[...] = proj(bc[0:1, 2:3] * s0_ref[...] + bc[0:1, 3:4] * s1_ref[...])


def _zpk(e0, e1, s0, s1, bcoef, Wp1, bp1, Wp2, bp2):
    br = 512
    return pl.pallas_call(
        _zp_body,
        grid=(NU // br,),
        in_specs=[
            pl.BlockSpec((br, H), lambda i: (i, 0)),
            pl.BlockSpec((br, H), lambda i: (i, 0)),
            pl.BlockSpec((br, H), lambda i: (i, 0)),
            pl.BlockSpec((br, H), lambda i: (i, 0)),
            pl.BlockSpec((1, 4), lambda i: (0, 0)),
            pl.BlockSpec((H, H), lambda i: (0, 0)),
            pl.BlockSpec((1, H), lambda i: (0, 0)),
            pl.BlockSpec((H, H), lambda i: (0, 0)),
            pl.BlockSpec((1, H), lambda i: (0, 0)),
        ],
        out_specs=[
            pl.BlockSpec((br, H), lambda i: (i, 0)),
            pl.BlockSpec((br, H), lambda i: (i, 0)),
        ],
        out_shape=[
            jax.ShapeDtypeStruct((NU, H), jnp.float32),
            jax.ShapeDtypeStruct((NU, H), jnp.float32),
        ],
    )(e0, e1, s0, s1, bcoef, Wp1, bp1.reshape(1, H), Wp2, bp2.reshape(1, H))


def _con_body(znm_i, zns_j, zns_i, znm_j, pos_ref, out_ref, rsA, aA, rsB, aB):
    j = pl.program_id(1)

    @pl.when(j == 0)
    def _():
        z = jnp.zeros_like(rsA[...])
        rsA[...] = z
        aA[...] = z
        rsB[...] = z
        aB[...] = z

    pos = pos_ref[...]
    dn = (((1,), (1,)), ((), ()))
    ma = jnp.exp(lax.dot_general(znm_i[...], zns_j[...], dn,
                                 preferred_element_type=jnp.float32)
                 * (1.0 / TAU))
    mb = jnp.exp(lax.dot_general(zns_i[...], znm_j[...], dn,
                                 preferred_element_type=jnp.float32)
                 * (1.0 / TAU))
    rsA[...] += jnp.sum(ma, axis=1, keepdims=True)
    aA[...] += jnp.sum(ma * pos, axis=1, keepdims=True)
    rsB[...] += jnp.sum(mb, axis=1, keepdims=True)
    aB[...] += jnp.sum(mb * pos, axis=1, keepdims=True)

    @pl.when(j == pl.num_programs(1) - 1)
    def _():
        vals = jnp.concatenate([
            jnp.sum(jnp.log(aA[...]), axis=0, keepdims=True),
            jnp.sum(jnp.log(rsA[...] + 1e-8), axis=0, keepdims=True),
            jnp.sum(jnp.log(aB[...]), axis=0, keepdims=True),
            jnp.sum(jnp.log(rsB[...] + 1e-8), axis=0, keepdims=True),
            jnp.zeros((1, 4), jnp.float32),
        ], axis=1)
        out_ref[...] = vals.reshape(1, 1, 8)


def _conk(znm, zns, pos):
    bb = 512
    nb = NU // bb
    return pl.pallas_call(
        _con_body,
        grid=(nb, nb),
        in_specs=[
            pl.BlockSpec((bb, H), lambda i, j: (i, 0)),
            pl.BlockSpec((bb, H), lambda i, j: (j, 0)),
            pl.BlockSpec((bb, H), lambda i, j: (i, 0)),
            pl.BlockSpec((bb, H), lambda i, j: (j, 0)),
            pl.BlockSpec((bb, bb), lambda i, j: (i, j)),
        ],
        out_specs=pl.BlockSpec((1, 1, 8), lambda i, j: (i, 0, 0)),
        out_shape=jax.ShapeDtypeStruct((nb, 1, 8), jnp.float32),
        scratch_shapes=[pltpu.VMEM((bb, 1), jnp.float32)] * 4,
    )(znm, zns, znm, zns, pos)


def kernel(x_user, x_t1, x_t2, pos, edge_index_mp0, edge_index_mp1, nei_t1,
           nei_t2, Wmap0, bmap0, Wmap1, bmap1, Wmap2, bmap2, Wg0, bg0, alpha0,
           Wg1, bg1, alpha1, Wfc_mp, bfc_mp, att_mp, att_i0, att_i1, Wfc_sc,
           bfc_sc, att_sc, Wp1, bp1, Wp2, bp2):
    f32 = jnp.float32
    aL0 = att_i0[0, :H]
    aR0 = att_i0[0, H:]
    aL1 = att_i1[0, :H]
    aR1 = att_i1[0, H:]

    Wbig = jnp.concatenate([Wg0, Wg1, aL0[:, None], aL1[:, None],
                            jnp.zeros((H, UFW - 2 * H - 2), f32)], 1)
    eye = jnp.eye(H, dtype=f32)
    Wr1 = jnp.concatenate([eye, aR0[:, None], jnp.zeros((H, HP - H - 1), f32)], 1)
    Wr2 = jnp.concatenate([eye, aR1[:, None], jnp.zeros((H, HP - H - 1), f32)], 1)

    uf = _proj(x_user, Wmap0, bmap0, Wbig)
    h1p = _proj(x_t1, Wmap1, bmap1, Wr1)
    h2p = _proj(x_t2, Wmap2, bmap2, Wr2)

    src0 = edge_index_mp0[0].astype(jnp.int32)
    dst0 = edge_index_mp0[1].astype(jnp.int32)
    src1 = edge_index_mp1[0].astype(jnp.int32)
    dst1 = edge_index_mp1[1].astype(jnp.int32)

    degp = _sc_deg(dst0, dst1)
    deg2 = degp.sum(0).reshape(2, NU).T
    y0, y1, di = _yk(deg2, uf)

    gout = _sc_segsum(y0, y1,
                      src0.reshape(E // 128, 128), dst0.reshape(E // 128, 128),
                      src1.reshape(E // 128, 128), dst1.reshape(E // 128, 128))

    bg = jnp.stack([bg0, bg1], 0)
    al = jnp.stack([alpha0, alpha1], 1)
    e0, e1 = _ek(gout, y0, y1, di, bg, al)

    s0r = _sc_intra(h1p, nei_t1.astype(jnp.int32).reshape(NU * S1 // 128, 128),
                    uf, S1, 0)
    s1r = _sc_intra(h2p, nei_t2.astype(jnp.int32).reshape(NU * S2 // 128, 128),
                    uf, S2, 1)

    s0, s1, ts = _semk(e0, e1, s0r, s1r, Wfc_mp, bfc_mp, Wfc_sc, bfc_sc)
    t = ts[:4] / NU
    b_mp = jax.nn.softmax(att_mp[0] @ t[0:2].T)
    b_sc = jax.nn.softmax(att_sc[0] @ t[2:4].T)
    bcoef = jnp.concatenate([b_mp, b_sc]).reshape(1, 4)

    znm, zns = _zpk(e0, e1, s0, s1, bcoef, Wp1, bp1, Wp2, bp2)
    part = _conk(znm, zns, pos)
    sla, slra, slb, slrb = (part[:, 0, 0].sum(), part[:, 0, 1].sum(),
                            part[:, 0, 2].sum(), part[:, 0, 3].sum())
    lori_mp = -(sla - slra) / NU
    lori_sc = -(slb - slrb) / NU
    return LAM * lori_mp + (1.0 - LAM) * lori_sc


# speedup vs baseline: 22.3810x; 22.3810x over previous
import functools

import jax
import jax.numpy as jnp
from jax import lax
from jax.experimental import pallas as pl
from jax.experimental.pallas import tpu as pltpu
from jax.experimental.pallas import tpu_sc as plsc

NU, N1, N2, DIN, H, E = 4096, 8192, 8192, 128, 64, 131072
S1, S2 = 8, 4
TAU, LAM = 0.8, 0.5

NC, NS = 2, 16
NW = NC * NS
EPT = E // NW
HP = 128
UFW = 256


def _elu(x):
    return jnp.where(x > 0, x, jnp.exp(jnp.minimum(x, 0.0)) - 1.0)


def _proj_body(x_ref, wm_ref, bm_ref, wr_ref, o_ref):
    h = _elu(jnp.dot(x_ref[...], wm_ref[...], preferred_element_type=jnp.float32)
             + bm_ref[...])
    o_ref[...] = jnp.dot(h, wr_ref[...], preferred_element_type=jnp.float32)


def _proj(x, Wmap, bmap, Wright, br=512):
    n = x.shape[0]
    co = Wright.shape[1]
    return pl.pallas_call(
        _proj_body,
        grid=(n // br,),
        in_specs=[
            pl.BlockSpec((br, DIN), lambda i: (i, 0)),
            pl.BlockSpec((DIN, H), lambda i: (0, 0)),
            pl.BlockSpec((1, H), lambda i: (0, 0)),
            pl.BlockSpec((H, co), lambda i: (0, 0)),
        ],
        out_specs=pl.BlockSpec((br, co), lambda i: (i, 0)),
        out_shape=jax.ShapeDtypeStruct((n, co), jnp.float32),
    )(x, Wmap, bmap.reshape(1, H), Wright)


def _sc_deg(dst0, dst1):
    mesh = plsc.VectorSubcoreMesh(core_axis_name="c", subcore_axis_name="s")

    @functools.partial(
        pl.kernel, mesh=mesh,
        out_type=jax.ShapeDtypeStruct((NC, 2, NU), jnp.float32),
        compiler_params=pltpu.CompilerParams(needs_layout_passes=False),
        scratch_types=[
            pltpu.VMEM((EPT,), jnp.int32),
            pltpu.VMEM((NU,), jnp.float32),
            pltpu.VMEM((NU,), jnp.float32),
            pltpu.VMEM((NS, 256), jnp.float32),
            pltpu.VMEM((256,), jnp.float32),
            pltpu.VMEM_SHARED((NS, NU), jnp.float32),
            pltpu.VMEM_SHARED((NS, NU), jnp.float32),
        ],
    )
    def k(d0_hbm, d1_hbm, degp, dstv, h0, h1, red, degv, sh0, sh1):
        cid = lax.axis_index("c")
        sid = lax.axis_index("s")
        wid = cid * NS + sid
        z16 = jnp.zeros((16,), jnp.float32)

        def fz(i, _):
            h0[pl.ds(i * 16, 16)] = z16
            h1[pl.ds(i * 16, 16)] = z16
            return 0
        lax.fori_loop(0, NU // 16, fz, 0)

        ones = jnp.full((16,), 1.0, jnp.float32)
        for mp in range(2):
            d_hbm, hist = ((d0_hbm, h0), (d1_hbm, h1))[mp]
            pltpu.sync_copy(d_hbm.at[pl.ds(wid * EPT, EPT)], dstv)

            def fh(i, _):
                idx = dstv[pl.ds(i * 16, 16)]
                plsc.addupdate_scatter(hist, [idx], ones)
                return 0
            lax.fori_loop(0, EPT // 16, fh, 0)

        pltpu.sync_copy(h0, sh0.at[sid])
        pltpu.sync_copy(h1, sh1.at[sid])
        plsc.subcore_barrier()

        for mp in range(2):
            sh = (sh0, sh1)[mp]
            pltpu.sync_copy(sh.at[:, pl.ds(sid * 256, 256)], red)
            for c in range(256 // 16):
                acc = red[0, pl.ds(c * 16, 16)]
                for t in range(1, NS):
                    acc = acc + red[t, pl.ds(c * 16, 16)]
                degv[pl.ds(c * 16, 16)] = acc
            pltpu.sync_copy(degv, degp.at[cid, mp, pl.ds(sid * 256, 256)])

    return k(dst0, dst1)


def _sc_segsum(y01, src0, dst0, src1, dst1):
    mesh = plsc.VectorSubcoreMesh(core_axis_name="c", subcore_axis_name="s")
    cpt = EPT // 128

    @functools.partial(
        pl.kernel, mesh=mesh,
        out_type=jax.ShapeDtypeStruct((NC, 2, NU, 128), jnp.float32),
        compiler_params=pltpu.CompilerParams(needs_layout_passes=False),
        scratch_types=[
            pltpu.VMEM((cpt, 128), jnp.int32),
            pltpu.VMEM((cpt, 128), jnp.int32),
            pltpu.VMEM((128, 128), jnp.float32),
            pltpu.VMEM((256, 128), jnp.float32),
            pltpu.VMEM_SHARED((NU, 128), jnp.float32),
            pltpu.SemaphoreType.DMA,
        ],
    )
    def k(y_hbm, s0_hbm, d0_hbm, s1_hbm, d1_hbm, gout,
          srcv, dstv, rows, zbuf, acc, sem):
        cid = lax.axis_index("c")
        sid = lax.axis_index("s")
        wid = cid * NS + sid
        z16 = jnp.zeros((16,), jnp.float32)

        def fz(i, _):
            for c in range(128 // 16):
                zbuf[i, pl.ds(c * 16, 16)] = z16
            return 0
        lax.fori_loop(0, 256, fz, 0)

        for mp in range(2):
            s_hbm, d_hbm = ((s0_hbm, d0_hbm), (s1_hbm, d1_hbm))[mp]
            pltpu.sync_copy(zbuf, acc.at[pl.ds(sid * 256, 256), :])
            plsc.subcore_barrier()
            pltpu.sync_copy(s_hbm.at[pl.ds(wid * cpt, cpt), :], srcv)
            pltpu.sync_copy(d_hbm.at[pl.ds(wid * cpt, cpt), :], dstv)

            def chunk(j, _):
                pltpu.async_copy(y_hbm.at[srcv.at[j]], rows, sem).wait()
                pltpu.sync_copy(rows, acc.at[dstv.at[j]], add=True)
                return 0
            lax.fori_loop(0, cpt, chunk, 0)
            plsc.subcore_barrier()
            pltpu.sync_copy(acc.at[pl.ds(sid * 256, 256), :],
                            gout.at[cid, mp, pl.ds(sid * 256, 256), :])
            plsc.subcore_barrier()

    return k(y01, src0, dst0, src1, dst1)


def _sc_intra(hp, neif, uf, S, tcol):
    mesh = plsc.VectorSubcoreMesh(core_axis_name="c", subcore_axis_name="s")
    P = NU // NW
    PB = 512 // S
    npass = P // PB

    @functools.partial(
        pl.kernel, mesh=mesh,
        out_type=jax.ShapeDtypeStruct((NU, 128), jnp.float32),
        compiler_params=pltpu.CompilerParams(needs_layout_passes=False),
        scratch_types=[
            pltpu.VMEM((P * S // 128, 128), jnp.int32),
            pltpu.VMEM((512, HP), jnp.float32),
            pltpu.VMEM((P, UFW), jnp.float32),
            pltpu.VMEM((P, 128), jnp.float32),
            pltpu.SemaphoreType.DMA,
        ],
    )
    def k(hp_hbm, nei_hbm, uf_hbm, sout, neiv, rows, ccv, outv, sem):
        cid = lax.axis_index("c")
        sid = lax.axis_index("s")
        wid = cid * NS + sid
        nch = P * S // 128
        pltpu.sync_copy(nei_hbm.at[pl.ds(wid * nch, nch), :], neiv)
        pltpu.sync_copy(uf_hbm.at[pl.ds(wid * P, P), :], ccv)

        lane = lax.iota(jnp.int32, 16)
        kk = jnp.minimum(lane, S - 1)
        col_d = jnp.full((16,), H, jnp.int32)
        col_c = jnp.full((16,), 2 * H + tcol, jnp.int32)
        cpp = 512 // 128

        for pv in range(npass):
            cps = [pltpu.async_copy(hp_hbm.at[neiv.at[pv * cpp + j]],
                                    rows.at[pl.ds(j * 128, 128), :], sem)
                   for j in range(cpp)]
            for cp in cps:
                cp.wait()

            def ub(u, _):
                base = u * S
                dv = plsc.load_gather(rows, [base + kk, col_d])
                cu = plsc.load_gather(ccv, [lane * 0 + (pv * PB + u), col_c])
                l = cu + dv
                l = jnp.where(l > 0, l, 0.01 * l)
                l = jnp.where(lane < S, l, -1e30)
                p = jnp.exp(l - jnp.max(l))
                den = jnp.sum(p)
                ws = [p[kq] for kq in range(S)]
                for f in range(H // 16):
                    acc = rows[base, pl.ds(f * 16, 16)] * ws[0]
                    for kq in range(1, S):
                        acc = acc + rows[base + kq, pl.ds(f * 16, 16)] * ws[kq]
                    outv[pv * PB + u, pl.ds(f * 16, 16)] = acc / den
                    outv[pv * PB + u, pl.ds(H + f * 16, 16)] = jnp.zeros(
                        (16,), jnp.float32)
                return 0
            lax.fori_loop(0, PB, ub, 0)
        pltpu.sync_copy(outv, sout.at[pl.ds(wid * P, P), :])

    return k(hp, neif, uf)


def _y_body(deg_ref, uf_ref, y_ref, di_ref):
    dinv = lax.rsqrt(deg_ref[...] + 1.0)
    di_ref[...] = dinv
    y_ref[...] = jnp.concatenate(
        [dinv[:, 0:1] * uf_ref[:, 0:H], dinv[:, 1:2] * uf_ref[:, H:2 * H]], 1)


def _yk(deg2, uf):
    br = 512
    return pl.pallas_call(
        _y_body,
        grid=(NU // br,),
        in_specs=[
            pl.BlockSpec((br, 2), lambda i: (i, 0)),
            pl.BlockSpec((br, UFW), lambda i: (i, 0)),
        ],
        out_specs=[
            pl.BlockSpec((br, 128), lambda i: (i, 0)),
            pl.BlockSpec((br, 2), lambda i: (i, 0)),
        ],
        out_shape=[
            jax.ShapeDtypeStruct((NU, 128), jnp.float32),
            jax.ShapeDtypeStruct((NU, 2), jnp.float32),
        ],
    )(deg2, uf)


def _e_body(g_ref, y_ref, di_ref, bg_ref, al_ref, e0_ref, e1_ref):
    g = g_ref[...]
    y = y_ref[...]
    di = di_ref[...]
    al = al_ref[...]
    bg = bg_ref[...]
    t0 = di[:, 0:1] * (g[0, 0, :, 0:H] + g[1, 0, :, 0:H] + y[:, 0:H]) + bg[0:1, :]
    e0_ref[...] = jnp.where(t0 > 0, t0, al[0:1, 0:1] * t0)
    t1 = di[:, 1:2] * (g[0, 1, :, H:2 * H] + g[1, 1, :, H:2 * H]
                       + y[:, H:2 * H]) + bg[1:2, :]
    e1_ref[...] = jnp.where(t1 > 0, t1, al[0:1, 1:2] * t1)


def _ek(gout, y01, di, bg, al):
    br = 512
    return pl.pallas_call(
        _e_body,
        grid=(NU // br,),
        in_specs=[
            pl.BlockSpec((NC, 2, br, 128), lambda i: (0, 0, i, 0)),
            pl.BlockSpec((br, 128), lambda i: (i, 0)),
            pl.BlockSpec((br, 2), lambda i: (i, 0)),
            pl.BlockSpec((2, H), lambda i: (0, 0)),
            pl.BlockSpec((1, 2), lambda i: (0, 0)),
        ],
        out_specs=[
            pl.BlockSpec((br, H), lambda i: (i, 0)),
            pl.BlockSpec((br, H), lambda i: (i, 0)),
        ],
        out_shape=[
            jax.ShapeDtypeStruct((NU, H), jnp.float32),
            jax.ShapeDtypeStruct((NU, H), jnp.float32),
        ],
    )(gout, y01, di, bg, al)


def _sem_body(e0_ref, e1_ref, s0r_ref, s1r_ref, wmp_ref, bmp_ref,
              wsc_ref, bsc_ref, s0_ref, s1_ref, ts_ref):
    i = pl.program_id(0)
    s0 = _elu(s0r_ref[:, 0:H])
    s1 = _elu(s1r_ref[:, 0:H])
    s0_ref[...] = s0
    s1_ref[...] = s1
    wmp = wmp_ref[...]
    bmp = bmp_ref[...]
    wsc = wsc_ref[...]
    bsc = bsc_ref[...]
    r0 = jnp.sum(jnp.tanh(jnp.dot(e0_ref[...], wmp,
                                  preferred_element_type=jnp.float32) + bmp),
                 axis=0, keepdims=True)
    r1 = jnp.sum(jnp.tanh(jnp.dot(e1_ref[...], wmp,
                                  preferred_element_type=jnp.float32) + bmp),
                 axis=0, keepdims=True)
    r2 = jnp.sum(jnp.tanh(jnp.dot(s0, wsc,
                                  preferred_element_type=jnp.float32) + bsc),
                 axis=0, keepdims=True)
    r3 = jnp.sum(jnp.tanh(jnp.dot(s1, wsc,
                                  preferred_element_type=jnp.float32) + bsc),
                 axis=0, keepdims=True)
    blk = jnp.concatenate([r0, r1, r2, r3, jnp.zeros((4, H), jnp.float32)], 0)

    @pl.when(i == 0)
    def _():
        ts_ref[...] = jnp.zeros((8, H), jnp.float32)

    ts_ref[...] += blk


def _semk(e0, e1, s0r, s1r, wmp, bmp, wsc, bsc):
    br = 512
    return pl.pallas_call(
        _sem_body,
        grid=(NU // br,),
        in_specs=[
            pl.BlockSpec((br, H), lambda i: (i, 0)),
            pl.BlockSpec((br, H), lambda i: (i, 0)),
            pl.BlockSpec((br, 128), lambda i: (i, 0)),
            pl.BlockSpec((br, 128), lambda i: (i, 0)),
            pl.BlockSpec((H, H), lambda i: (0, 0)),
            pl.BlockSpec((1, H), lambda i: (0, 0)),
            pl.BlockSpec((H, H), lambda i: (0, 0)),
            pl.BlockSpec((1, H), lambda i: (0, 0)),
        ],
        out_specs=[
            pl.BlockSpec((br, H), lambda i: (i, 0)),
            pl.BlockSpec((br, H), lambda i: (i, 0)),
            pl.BlockSpec((8, H), lambda i: (0, 0)),
        ],
        out_shape=[
            jax.ShapeDtypeStruct((NU, H), jnp.float32),
            jax.ShapeDtypeStruct((NU, H), jnp.float32),
            jax.ShapeDtypeStruct((8, H), jnp.float32),
        ],
    )(e0, e1, s0r, s1r, wmp, bmp.reshape(1, H), wsc, bsc.reshape(1, H))


def _zp_body(e0_ref, e1_ref, s0_ref, s1_ref, bc_ref, w1_ref, b1_ref,
             w2_ref, b2_ref, znm_ref, zns_ref):
    bc = bc_ref[...]
    w1 = w1_ref[...]
    b1 = b1_ref[...]
    w2 = w2_ref[...]
    b2 = b2_ref[...]

    def proj(z):
        zp = jnp.dot(_elu(jnp.dot(z, w1, preferred_element_type=jnp.float32)
                          + b1), w2, preferred_element_type=jnp.float32) + b2
        return zp * lax.rsqrt(jnp.sum(zp * zp, axis=1, keepdims=True))

    znm_ref[...] = proj(bc[0:1, 0:1] * e0_ref[...] + bc[0:1, 1:2] * e1_ref[...])
    zns_ref[...] = proj(bc[0:1, 2:3] * s0_ref[...] + bc[0:1, 3:4] * s1_ref[...])


def _zpk(e0, e1, s0, s1, bcoef, Wp1, bp1, Wp2, bp2):
    br = 512
    return pl.pallas_call(
        _zp_body,
        grid=(NU // br,),
        in_specs=[
            pl.BlockSpec((br, H), lambda i: (i, 0)),
            pl.BlockSpec((br, H), lambda i: (i, 0)),
            pl.BlockSpec((br, H), lambda i: (i, 0)),
            pl.BlockSpec((br, H), lambda i: (i, 0)),
            pl.BlockSpec((1, 4), lambda i: (0, 0)),
            pl.BlockSpec((H, H), lambda i: (0, 0)),
            pl.BlockSpec((1, H), lambda i: (0, 0)),
            pl.BlockSpec((H, H), lambda i: (0, 0)),
            pl.BlockSpec((1, H), lambda i: (0, 0)),
        ],
        out_specs=[
            pl.BlockSpec((br, H), lambda i: (i, 0)),
            pl.BlockSpec((br, H), lambda i: (i, 0)),
        ],
        out_shape=[
            jax.ShapeDtypeStruct((NU, H), jnp.float32),
            jax.ShapeDtypeStruct((NU, H), jnp.float32),
        ],
    )(e0, e1, s0, s1, bcoef, Wp1, bp1.reshape(1, H), Wp2, bp2.reshape(1, H))


def _con_body(znm_i, zns_j, zns_i, znm_j, pos_ref, out_ref, rsA, aA, rsB, aB):
    j = pl.program_id(1)

    @pl.when(j == 0)
    def _():
        z = jnp.zeros_like(rsA[...])
        rsA[...] = z
        aA[...] = z
        rsB[...] = z
        aB[...] = z

    pos = pos_ref[...]
    dn = (((1,), (1,)), ((), ()))
    ma = jnp.exp(lax.dot_general(znm_i[...], zns_j[...], dn,
                                 preferred_element_type=jnp.float32)
                 * (1.0 / TAU))
    mb = jnp.exp(lax.dot_general(zns_i[...], znm_j[...], dn,
                                 preferred_element_type=jnp.float32)
                 * (1.0 / TAU))
    rsA[...] += jnp.sum(ma, axis=1, keepdims=True)
    aA[...] += jnp.sum(ma * pos, axis=1, keepdims=True)
    rsB[...] += jnp.sum(mb, axis=1, keepdims=True)
    aB[...] += jnp.sum(mb * pos, axis=1, keepdims=True)

    @pl.when(j == pl.num_programs(1) - 1)
    def _():
        vals = jnp.concatenate([
            jnp.sum(jnp.log(aA[...]), axis=0, keepdims=True),
            jnp.sum(jnp.log(rsA[...] + 1e-8), axis=0, keepdims=True),
            jnp.sum(jnp.log(aB[...]), axis=0, keepdims=True),
            jnp.sum(jnp.log(rsB[...] + 1e-8), axis=0, keepdims=True),
            jnp.zeros((1, 4), jnp.float32),
        ], axis=1)
        out_ref[...] = vals.reshape(1, 1, 8)


def _conk(znm, zns, pos):
    bb = 512
    nb = NU // bb
    return pl.pallas_call(
        _con_body,
        grid=(nb, nb),
        in_specs=[
            pl.BlockSpec((bb, H), lambda i, j: (i, 0)),
            pl.BlockSpec((bb, H), lambda i, j: (j, 0)),
            pl.BlockSpec((bb, H), lambda i, j: (i, 0)),
            pl.BlockSpec((bb, H), lambda i, j: (j, 0)),
            pl.BlockSpec((bb, bb), lambda i, j: (i, j)),
        ],
        out_specs=pl.BlockSpec((1, 1, 8), lambda i, j: (i, 0, 0)),
        out_shape=jax.ShapeDtypeStruct((nb, 1, 8), jnp.float32),
        scratch_shapes=[pltpu.VMEM((bb, 1), jnp.float32)] * 4,
    )(znm, zns, znm, zns, pos)


def kernel(x_user, x_t1, x_t2, pos, edge_index_mp0, edge_index_mp1, nei_t1,
           nei_t2, Wmap0, bmap0, Wmap1, bmap1, Wmap2, bmap2, Wg0, bg0, alpha0,
           Wg1, bg1, alpha1, Wfc_mp, bfc_mp, att_mp, att_i0, att_i1, Wfc_sc,
           bfc_sc, att_sc, Wp1, bp1, Wp2, bp2):
    f32 = jnp.float32
    aL0 = att_i0[0, :H]
    aR0 = att_i0[0, H:]
    aL1 = att_i1[0, :H]
    aR1 = att_i1[0, H:]

    Wbig = jnp.concatenate([Wg0, Wg1, aL0[:, None], aL1[:, None],
                            jnp.zeros((H, UFW - 2 * H - 2), f32)], 1)
    eye = jnp.eye(H, dtype=f32)
    Wr1 = jnp.concatenate([eye, aR0[:, None], jnp.zeros((H, HP - H - 1), f32)], 1)
    Wr2 = jnp.concatenate([eye, aR1[:, None], jnp.zeros((H, HP - H - 1), f32)], 1)


    uf = _proj(x_user, Wmap0, bmap0, Wbig)
    h1p = _proj(x_t1, Wmap1, bmap1, Wr1)
    h2p = _proj(x_t2, Wmap2, bmap2, Wr2)

    src0 = edge_index_mp0[0].astype(jnp.int32)
    dst0 = edge_index_mp0[1].astype(jnp.int32)
    src1 = edge_index_mp1[0].astype(jnp.int32)
    dst1 = edge_index_mp1[1].astype(jnp.int32)

    degp = _sc_deg(dst0, dst1)
    deg2 = degp.sum(0).T
    y01, di = _yk(deg2, uf)

    gout = _sc_segsum(y01,
                      src0.reshape(E // 128, 128), dst0.reshape(E // 128, 128),
                      src1.reshape(E // 128, 128), dst1.reshape(E // 128, 128))

    bg = jnp.stack([bg0, bg1], 0)
    al = jnp.stack([alpha0, alpha1], 1)
    e0, e1 = _ek(gout, y01, di, bg, al)

    s0r = _sc_intra(h1p, nei_t1.astype(jnp.int32).reshape(NU * S1 // 128, 128),
                    uf, S1, 0)
    s1r = _sc_intra(h2p, nei_t2.astype(jnp.int32).reshape(NU * S2 // 128, 128),
                    uf, S2, 1)

    s0, s1, ts = _semk(e0, e1, s0r, s1r, Wfc_mp, bfc_mp, Wfc_sc, bfc_sc)
    t = ts[:4] / NU
    b_mp = jax.nn.softmax(att_mp[0] @ t[0:2].T)
    b_sc = jax.nn.softmax(att_sc[0] @ t[2:4].T)
    bcoef = jnp.concatenate([b_mp, b_sc]).reshape(1, 4)

    znm, zns = _zpk(e0, e1, s0, s1, bcoef, Wp1, bp1, Wp2, bp2)
    part = _conk(znm, zns, pos)
    sla, slra, slb, slrb = (part[:, 0, 0].sum(), part[:, 0, 1].sum(),
                            part[:, 0, 2].sum(), part[:, 0, 3].sum())
    lori_mp = -(sla - slra) / NU
    lori_sc = -(slb - slrb) / NU
    return LAM * lori_mp + (1.0 - LAM) * lori_sc


# speedup vs baseline: 25.7946x; 1.1525x over previous
import functools

import jax
import jax.numpy as jnp
from jax import lax
from jax.experimental import pallas as pl
from jax.experimental.pallas import tpu as pltpu
from jax.experimental.pallas import tpu_sc as plsc

NU, N1, N2, DIN, H, E = 4096, 8192, 8192, 128, 64, 131072
S1, S2 = 8, 4
TAU, LAM = 0.8, 0.5

NC, NS = 2, 16
NW = NC * NS
EPT = E // NW
HP = 128
UFW = 256


def _elu(x):
    return jnp.where(x > 0, x, jnp.exp(jnp.minimum(x, 0.0)) - 1.0)


def _proj_body(x_ref, wm_ref, bm_ref, wr_ref, o_ref):
    h = _elu(jnp.dot(x_ref[...], wm_ref[...], preferred_element_type=jnp.float32)
             + bm_ref[...])
    o_ref[...] = jnp.dot(h, wr_ref[...], preferred_element_type=jnp.float32)


def _proj(x, Wmap, bmap, Wright, br=512):
    n = x.shape[0]
    co = Wright.shape[1]
    return pl.pallas_call(
        _proj_body,
        grid=(n // br,),
        in_specs=[
            pl.BlockSpec((br, DIN), lambda i: (i, 0)),
            pl.BlockSpec((DIN, H), lambda i: (0, 0)),
            pl.BlockSpec((1, H), lambda i: (0, 0)),
            pl.BlockSpec((H, co), lambda i: (0, 0)),
        ],
        out_specs=pl.BlockSpec((br, co), lambda i: (i, 0)),
        out_shape=jax.ShapeDtypeStruct((n, co), jnp.float32),
    )(x, Wmap, bmap.reshape(1, H), Wright)


def _sc_deg(dst0, dst1):
    mesh = plsc.VectorSubcoreMesh(core_axis_name="c", subcore_axis_name="s")

    @functools.partial(
        pl.kernel, mesh=mesh,
        out_type=jax.ShapeDtypeStruct((NC, 2, NU), jnp.float32),
        compiler_params=pltpu.CompilerParams(needs_layout_passes=False),
        scratch_types=[
            pltpu.VMEM((EPT,), jnp.int32),
            pltpu.VMEM((NU,), jnp.float32),
            pltpu.VMEM((NU,), jnp.float32),
            pltpu.VMEM((NS, 256), jnp.float32),
            pltpu.VMEM((256,), jnp.float32),
            pltpu.VMEM_SHARED((NS, NU), jnp.float32),
            pltpu.VMEM_SHARED((NS, NU), jnp.float32),
        ],
    )
    def k(d0_hbm, d1_hbm, degp, dstv, h0, h1, red, degv, sh0, sh1):
        cid = lax.axis_index("c")
        sid = lax.axis_index("s")
        wid = cid * NS + sid
        z16 = jnp.zeros((16,), jnp.float32)

        def fz(i, _):
            h0[pl.ds(i * 16, 16)] = z16
            h1[pl.ds(i * 16, 16)] = z16
            return 0
        lax.fori_loop(0, NU // 16, fz, 0)

        ones = jnp.full((16,), 1.0, jnp.float32)
        for mp in range(2):
            d_hbm, hist = ((d0_hbm, h0), (d1_hbm, h1))[mp]
            pltpu.sync_copy(d_hbm.at[pl.ds(wid * EPT, EPT)], dstv)

            def fh(i, _):
                idx = dstv[pl.ds(i * 16, 16)]
                plsc.addupdate_scatter(hist, [idx], ones)
                return 0
            lax.fori_loop(0, EPT // 16, fh, 0)

        pltpu.sync_copy(h0, sh0.at[sid])
        pltpu.sync_copy(h1, sh1.at[sid])
        plsc.subcore_barrier()

        for mp in range(2):
            sh = (sh0, sh1)[mp]
            pltpu.sync_copy(sh.at[:, pl.ds(sid * 256, 256)], red)
            for c in range(256 // 16):
                acc = red[0, pl.ds(c * 16, 16)]
                for t in range(1, NS):
                    acc = acc + red[t, pl.ds(c * 16, 16)]
                degv[pl.ds(c * 16, 16)] = acc
            pltpu.sync_copy(degv, degp.at[cid, mp, pl.ds(sid * 256, 256)])

    return k(dst0, dst1)


def _sc_segsum(y01, src0, dst0, src1, dst1):
    mesh = plsc.VectorSubcoreMesh(core_axis_name="c", subcore_axis_name="s")
    cpt = EPT // 128

    @functools.partial(
        pl.kernel, mesh=mesh,
        out_type=jax.ShapeDtypeStruct((NC, 2, NU, 128), jnp.float32),
        compiler_params=pltpu.CompilerParams(needs_layout_passes=False),
        scratch_types=[
            pltpu.VMEM((2 * cpt, 128), jnp.int32),
            pltpu.VMEM((2 * cpt, 128), jnp.int32),
            pltpu.VMEM((128, 128), jnp.float32),
            pltpu.VMEM((128, 128), jnp.float32),
            pltpu.VMEM((256, 128), jnp.float32),
            pltpu.VMEM_SHARED((NU, 128), jnp.float32),
            pltpu.SemaphoreType.DMA,
            pltpu.SemaphoreType.DMA,
        ],
    )
    def k(y_hbm, s0_hbm, d0_hbm, s1_hbm, d1_hbm, gout,
          srcv, dstv, ra, rb, zbuf, acc, sem_a, sem_b):
        cid = lax.axis_index("c")
        sid = lax.axis_index("s")
        wid = cid * NS + sid
        z16 = jnp.zeros((16,), jnp.float32)

        def fz(i, _):
            for c in range(128 // 16):
                zbuf[i, pl.ds(c * 16, 16)] = z16
            return 0
        lax.fori_loop(0, 256, fz, 0)

        pltpu.sync_copy(s0_hbm.at[pl.ds(wid * cpt, cpt), :],
                        srcv.at[pl.ds(0, cpt), :])
        pltpu.sync_copy(s1_hbm.at[pl.ds(wid * cpt, cpt), :],
                        srcv.at[pl.ds(cpt, cpt), :])
        pltpu.sync_copy(d0_hbm.at[pl.ds(wid * cpt, cpt), :],
                        dstv.at[pl.ds(0, cpt), :])
        pltpu.sync_copy(d1_hbm.at[pl.ds(wid * cpt, cpt), :],
                        dstv.at[pl.ds(cpt, cpt), :])

        bufs = (ra, rb)
        for mp in range(2):
            pltpu.sync_copy(zbuf, acc.at[pl.ds(sid * 256, 256), :])
            plsc.subcore_barrier()
            base = mp * cpt
            pending = pltpu.async_copy(y_hbm.at[srcv.at[base]], bufs[0], sem_a)
            for t in range(cpt):
                cur = pending
                curbuf = bufs[t % 2]
                if t + 1 < cpt:
                    pending = pltpu.async_copy(
                        y_hbm.at[srcv.at[base + t + 1]], bufs[(t + 1) % 2],
                        (sem_a, sem_b)[(t + 1) % 2])
                cur.wait()
                pltpu.sync_copy(curbuf, acc.at[dstv.at[base + t]], add=True)
            plsc.subcore_barrier()
            pltpu.sync_copy(acc.at[pl.ds(sid * 256, 256), :],
                            gout.at[cid, mp, pl.ds(sid * 256, 256), :])
            plsc.subcore_barrier()

    return k(y01, src0, dst0, src1, dst1)


def _sc_intra(hp, neif, uf, S, tcol):
    mesh = plsc.VectorSubcoreMesh(core_axis_name="c", subcore_axis_name="s")
    P = NU // NW
    PB = 512 // S
    npass = P // PB

    @functools.partial(
        pl.kernel, mesh=mesh,
        out_type=jax.ShapeDtypeStruct((NU, 128), jnp.float32),
        compiler_params=pltpu.CompilerParams(needs_layout_passes=False),
        scratch_types=[
            pltpu.VMEM((P * S // 128, 128), jnp.int32),
            pltpu.VMEM((512, HP), jnp.float32),
            pltpu.VMEM((P, UFW), jnp.float32),
            pltpu.VMEM((P, 128), jnp.float32),
            pltpu.SemaphoreType.DMA,
        ],
    )
    def k(hp_hbm, nei_hbm, uf_hbm, sout, neiv, rows, ccv, outv, sem):
        cid = lax.axis_index("c")
        sid = lax.axis_index("s")
        wid = cid * NS + sid
        nch = P * S // 128
        pltpu.sync_copy(nei_hbm.at[pl.ds(wid * nch, nch), :], neiv)
        pltpu.sync_copy(uf_hbm.at[pl.ds(wid * P, P), :], ccv)

        lane = lax.iota(jnp.int32, 16)
        kk = jnp.minimum(lane, S - 1)
        col_d = jnp.full((16,), H, jnp.int32)
        col_c = jnp.full((16,), 2 * H + tcol, jnp.int32)
        cpp = 512 // 128

        for pv in range(npass):
            cps = [pltpu.async_copy(hp_hbm.at[neiv.at[pv * cpp + j]],
                                    rows.at[pl.ds(j * 128, 128), :], sem)
                   for j in range(cpp)]
            for cp in cps:
                cp.wait()

            def ub(u, _):
                base = u * S
                dv = plsc.load_gather(rows, [base + kk, col_d])
                cu = plsc.load_gather(ccv, [lane * 0 + (pv * PB + u), col_c])
                l = cu + dv
                l = jnp.where(l > 0, l, 0.01 * l)
                l = jnp.where(lane < S, l, -1e30)
                p = jnp.exp(l - jnp.max(l))
                den = jnp.sum(p)
                ws = [p[kq] for kq in range(S)]
                for f in range(H // 16):
                    acc = rows[base, pl.ds(f * 16, 16)] * ws[0]
                    for kq in range(1, S):
                        acc = acc + rows[base + kq, pl.ds(f * 16, 16)] * ws[kq]
                    outv[pv * PB + u, pl.ds(f * 16, 16)] = acc / den
                    outv[pv * PB + u, pl.ds(H + f * 16, 16)] = jnp.zeros(
                        (16,), jnp.float32)
                return 0
            lax.fori_loop(0, PB, ub, 0)
        pltpu.sync_copy(outv, sout.at[pl.ds(wid * P, P), :])

    return k(hp, neif, uf)


def _y_body(deg_ref, uf_ref, y_ref, di_ref):
    dinv = lax.rsqrt(deg_ref[...] + 1.0)
    di_ref[...] = dinv
    y_ref[...] = jnp.concatenate(
        [dinv[:, 0:1] * uf_ref[:, 0:H], dinv[:, 1:2] * uf_ref[:, H:2 * H]], 1)


def _yk(deg2, uf):
    br = 512
    return pl.pallas_call(
        _y_body,
        grid=(NU // br,),
        in_specs=[
            pl.BlockSpec((br, 2), lambda i: (i, 0)),
            pl.BlockSpec((br, UFW), lambda i: (i, 0)),
        ],
        out_specs=[
            pl.BlockSpec((br, 128), lambda i: (i, 0)),
            pl.BlockSpec((br, 2), lambda i: (i, 0)),
        ],
        out_shape=[
            jax.ShapeDtypeStruct((NU, 128), jnp.float32),
            jax.ShapeDtypeStruct((NU, 2), jnp.float32),
        ],
    )(deg2, uf)


def _epi_body(g_ref, y_ref, di_ref, bg_ref, al_ref, s0r_ref, s1r_ref,
              wmp_ref, bmp_ref, wsc_ref, bsc_ref, e0_ref, e1_ref, ts_ref):
    i = pl.program_id(0)
    g = g_ref[...]
    y = y_ref[...]
    di = di_ref[...]
    al = al_ref[...]
    bg = bg_ref[...]
    t0 = di[:, 0:1] * (g[0, 0, :, 0:H] + g[1, 0, :, 0:H] + y[:, 0:H]) + bg[0:1, :]
    e0 = jnp.where(t0 > 0, t0, al[0:1, 0:1] * t0)
    t1 = di[:, 1:2] * (g[0, 1, :, H:2 * H] + g[1, 1, :, H:2 * H]
                       + y[:, H:2 * H]) + bg[1:2, :]
    e1 = jnp.where(t1 > 0, t1, al[0:1, 1:2] * t1)
    e0_ref[...] = e0
    e1_ref[...] = e1
    s0 = _elu(s0r_ref[:, 0:H])
    s1 = _elu(s1r_ref[:, 0:H])
    wmp = wmp_ref[...]
    bmp = bmp_ref[...]
    wsc = wsc_ref[...]
    bsc = bsc_ref[...]
    f32 = jnp.float32
    r0 = jnp.sum(jnp.tanh(jnp.dot(e0, wmp, preferred_element_type=f32) + bmp),
                 axis=0, keepdims=True)
    r1 = jnp.sum(jnp.tanh(jnp.dot(e1, wmp, preferred_element_type=f32) + bmp),
                 axis=0, keepdims=True)
    r2 = jnp.sum(jnp.tanh(jnp.dot(s0, wsc, preferred_element_type=f32) + bsc),
                 axis=0, keepdims=True)
    r3 = jnp.sum(jnp.tanh(jnp.dot(s1, wsc, preferred_element_type=f32) + bsc),
                 axis=0, keepdims=True)
    blk = jnp.concatenate([r0, r1, r2, r3, jnp.zeros((4, H), f32)], 0)

    @pl.when(i == 0)
    def _():
        ts_ref[...] = jnp.zeros((8, H), f32)

    ts_ref[...] += blk


def _epik(gout, y01, di, bg, al, s0r, s1r, wmp, bmp, wsc, bsc):
    br = 512
    return pl.pallas_call(
        _epi_body,
        grid=(NU // br,),
        in_specs=[
            pl.BlockSpec((NC, 2, br, 128), lambda i: (0, 0, i, 0)),
            pl.BlockSpec((br, 128), lambda i: (i, 0)),
            pl.BlockSpec((br, 2), lambda i: (i, 0)),
            pl.BlockSpec((2, H), lambda i: (0, 0)),
            pl.BlockSpec((1, 2), lambda i: (0, 0)),
            pl.BlockSpec((br, 128), lambda i: (i, 0)),
            pl.BlockSpec((br, 128), lambda i: (i, 0)),
            pl.BlockSpec((H, H), lambda i: (0, 0)),
            pl.BlockSpec((1, H), lambda i: (0, 0)),
            pl.BlockSpec((H, H), lambda i: (0, 0)),
            pl.BlockSpec((1, H), lambda i: (0, 0)),
        ],
        out_specs=[
            pl.BlockSpec((br, H), lambda i: (i, 0)),
            pl.BlockSpec((br, H), lambda i: (i, 0)),
            pl.BlockSpec((8, H), lambda i: (0, 0)),
        ],
        out_shape=[
            jax.ShapeDtypeStruct((NU, H), jnp.float32),
            jax.ShapeDtypeStruct((NU, H), jnp.float32),
            jax.ShapeDtypeStruct((8, H), jnp.float32),
        ],
    )(gout, y01, di, bg, al, s0r, s1r, wmp, bmp.reshape(1, H),
      wsc, bsc.reshape(1, H))


def _zp_body(e0_ref, e1_ref, s0r_ref, s1r_ref, bc_ref, w1_ref, b1_ref,
             w2_ref, b2_ref, znm_ref, zns_ref):
    bc = bc_ref[...]
    w1 = w1_ref[...]
    b1 = b1_ref[...]
    w2 = w2_ref[...]
    b2 = b2_ref[...]

    def proj(z):
        zp = jnp.dot(_elu(jnp.dot(z, w1, preferred_element_type=jnp.float32)
                          + b1), w2, preferred_element_type=jnp.float32) + b2
        return zp * lax.rsqrt(jnp.sum(zp * zp, axis=1, keepdims=True))

    znm_ref[...] = proj(bc[0:1, 0:1] * e0_ref[...] + bc[0:1, 1:2] * e1_ref[...])
    zns_ref[...] = proj(bc[0:1, 2:3] * _elu(s0r_ref[:, 0:H])
                        + bc[0:1, 3:4] * _elu(s1r_ref[:, 0:H]))


def _zpk(e0, e1, s0r, s1r, bcoef, Wp1, bp1, Wp2, bp2):
    br = 512
    return pl.pallas_call(
        _zp_body,
        grid=(NU // br,),
        in_specs=[
            pl.BlockSpec((br, H), lambda i: (i, 0)),
            pl.BlockSpec((br, H), lambda i: (i, 0)),
            pl.BlockSpec((br, 128), lambda i: (i, 0)),
            pl.BlockSpec((br, 128), lambda i: (i, 0)),
            pl.BlockSpec((1, 4), lambda i: (0, 0)),
            pl.BlockSpec((H, H), lambda i: (0, 0)),
            pl.BlockSpec((1, H), lambda i: (0, 0)),
            pl.BlockSpec((H, H), lambda i: (0, 0)),
            pl.BlockSpec((1, H), lambda i: (0, 0)),
        ],
        out_specs=[
            pl.BlockSpec((br, H), lambda i: (i, 0)),
            pl.BlockSpec((br, H), lambda i: (i, 0)),
        ],
        out_shape=[
            jax.ShapeDtypeStruct((NU, H), jnp.float32),
            jax.ShapeDtypeStruct((NU, H), jnp.float32),
        ],
    )(e0, e1, s0r, s1r, bcoef, Wp1, bp1.reshape(1, H), Wp2, bp2.reshape(1, H))


def _con_body(znm_i, zns_j, zns_i, znm_j, pos_ref, out_ref, rsA, aA, rsB, aB):
    j = pl.program_id(1)

    @pl.when(j == 0)
    def _():
        z = jnp.zeros_like(rsA[...])
        rsA[...] = z
        aA[...] = z
        rsB[...] = z
        aB[...] = z

    pos = pos_ref[...]
    dn = (((1,), (1,)), ((), ()))
    ma = jnp.exp(lax.dot_general(znm_i[...], zns_j[...], dn,
                                 preferred_element_type=jnp.float32)
                 * (1.0 / TAU))
    mb = jnp.exp(lax.dot_general(zns_i[...], znm_j[...], dn,
                                 preferred_element_type=jnp.float32)
                 * (1.0 / TAU))
    rsA[...] += jnp.sum(ma, axis=1, keepdims=True)
    aA[...] += jnp.sum(ma * pos, axis=1, keepdims=True)
    rsB[...] += jnp.sum(mb, axis=1, keepdims=True)
    aB[...] += jnp.sum(mb * pos, axis=1, keepdims=True)

    @pl.when(j == pl.num_programs(1) - 1)
    def _():
        vals = jnp.concatenate([
            jnp.sum(jnp.log(aA[...]), axis=0, keepdims=True),
            jnp.sum(jnp.log(rsA[...] + 1e-8), axis=0, keepdims=True),
            jnp.sum(jnp.log(aB[...]), axis=0, keepdims=True),
            jnp.sum(jnp.log(rsB[...] + 1e-8), axis=0, keepdims=True),
            jnp.zeros((1, 4), jnp.float32),
        ], axis=1)
        out_ref[...] = vals.reshape(1, 1, 8)


def _conk(znm, zns, pos):
    bb = 512
    nb = NU // bb
    return pl.pallas_call(
        _con_body,
        grid=(nb, nb),
        in_specs=[
            pl.BlockSpec((bb, H), lambda i, j: (i, 0)),
            pl.BlockSpec((bb, H), lambda i, j: (j, 0)),
            pl.BlockSpec((bb, H), lambda i, j: (i, 0)),
            pl.BlockSpec((bb, H), lambda i, j: (j, 0)),
            pl.BlockSpec((bb, bb), lambda i, j: (i, j)),
        ],
        out_specs=pl.BlockSpec((1, 1, 8), lambda i, j: (i, 0, 0)),
        out_shape=jax.ShapeDtypeStruct((nb, 1, 8), jnp.float32),
        scratch_shapes=[pltpu.VMEM((bb, 1), jnp.float32)] * 4,
    )(znm, zns, znm, zns, pos)


def kernel(x_user, x_t1, x_t2, pos, edge_index_mp0, edge_index_mp1, nei_t1,
           nei_t2, Wmap0, bmap0, Wmap1, bmap1, Wmap2, bmap2, Wg0, bg0, alpha0,
           Wg1, bg1, alpha1, Wfc_mp, bfc_mp, att_mp, att_i0, att_i1, Wfc_sc,
           bfc_sc, att_sc, Wp1, bp1, Wp2, bp2):
    f32 = jnp.float32
    aL0 = att_i0[0, :H]
    aR0 = att_i0[0, H:]
    aL1 = att_i1[0, :H]
    aR1 = att_i1[0, H:]

    Wbig = jnp.concatenate([Wg0, Wg1, aL0[:, None], aL1[:, None],
                            jnp.zeros((H, UFW - 2 * H - 2), f32)], 1)
    eye = jnp.eye(H, dtype=f32)
    Wr1 = jnp.concatenate([eye, aR0[:, None], jnp.zeros((H, HP - H - 1), f32)], 1)
    Wr2 = jnp.concatenate([eye, aR1[:, None], jnp.zeros((H, HP - H - 1), f32)], 1)


    uf = _proj(x_user, Wmap0, bmap0, Wbig)
    h1p = _proj(x_t1, Wmap1, bmap1, Wr1)
    h2p = _proj(x_t2, Wmap2, bmap2, Wr2)

    src0 = edge_index_mp0[0].astype(jnp.int32)
    dst0 = edge_index_mp0[1].astype(jnp.int32)
    src1 = edge_index_mp1[0].astype(jnp.int32)
    dst1 = edge_index_mp1[1].astype(jnp.int32)

    degp = _sc_deg(dst0, dst1)
    deg2 = degp.sum(0).T
    y01, di = _yk(deg2, uf)

    gout = _sc_segsum(y01,
                      src0.reshape(E // 128, 128), dst0.reshape(E // 128, 128),
                      src1.reshape(E // 128, 128), dst1.reshape(E // 128, 128))

    s0r = _sc_intra(h1p, nei_t1.astype(jnp.int32).reshape(NU * S1 // 128, 128),
                    uf, S1, 0)
    s1r = _sc_intra(h2p, nei_t2.astype(jnp.int32).reshape(NU * S2 // 128, 128),
                    uf, S2, 1)

    bg = jnp.stack([bg0, bg1], 0)
    al = jnp.stack([alpha0, alpha1], 1)
    e0, e1, ts = _epik(gout, y01, di, bg, al, s0r, s1r,
                       Wfc_mp, bfc_mp, Wfc_sc, bfc_sc)
    t = ts[:4] / NU
    b_mp = jax.nn.softmax(att_mp[0] @ t[0:2].T)
    b_sc = jax.nn.softmax(att_sc[0] @ t[2:4].T)
    bcoef = jnp.concatenate([b_mp, b_sc]).reshape(1, 4)

    znm, zns = _zpk(e0, e1, s0r, s1r, bcoef, Wp1, bp1, Wp2, bp2)
    part = _conk(znm, zns, pos)
    sla, slra, slb, slrb = (part[:, 0, 0].sum(), part[:, 0, 1].sum(),
                            part[:, 0, 2].sum(), part[:, 0, 3].sum())
    lori_mp = -(sla - slra) / NU
    lori_sc = -(slb - slrb) / NU
    return LAM * lori_mp + (1.0 - LAM) * lori_sc
